# Initial kernel scaffold; baseline (speedup 1.0000x reference)
import jax
import jax.numpy as jnp
from jax.experimental import pallas as pl


def kernel(x, edge_index, edge_weight, W, b):
    raise NotImplementedError("write your pallas kernel here")


# speedup vs baseline: 5.2059x; 5.2059x over previous
import functools

import jax
import jax.numpy as jnp
from jax import lax
from jax.experimental import pallas as pl
from jax.experimental.pallas import tpu as pltpu
from jax.experimental.pallas import tpu_sc as plsc

NC = 2
NS = 16
NW = NC * NS
CH = 128


def _sc_aggregate(x, src3, dst3, w3, n_acc, d_feat, n_chunks):
    rows_per_sub = n_acc // NS

    mesh = plsc.VectorSubcoreMesh(core_axis_name="c", subcore_axis_name="s")

    @functools.partial(
        pl.kernel,
        out_type=jax.ShapeDtypeStruct((NC, n_acc, d_feat), jnp.float32),
        mesh=mesh,
        scratch_types=[
            pltpu.VMEM((n_chunks, CH), jnp.int32),
            pltpu.VMEM((n_chunks, CH), jnp.int32),
            pltpu.VMEM((n_chunks, CH), jnp.float32),
            pltpu.VMEM((CH, 128), jnp.float32),
            pltpu.VMEM_SHARED((n_acc, 128), jnp.float32),
            pltpu.SemaphoreType.DMA,
        ],
    )
    def body(x_hbm, src_hbm, dst_hbm, w_hbm, out_hbm,
             src_v, dst_v, w_v, rows_v, acc_sh, sem):
        c = lax.axis_index("c")
        s = lax.axis_index("s")
        wid = s * NC + c

        def zrow(i, _):
            for cc in range(8):
                rows_v[i, pl.ds(cc * 16, 16)] = jnp.zeros((16,), jnp.float32)
            return 0
        lax.fori_loop(0, CH, zrow, 0)
        for i in range(rows_per_sub // CH):
            pltpu.sync_copy(rows_v, acc_sh.at[pl.ds(s * rows_per_sub + i * CH, CH)])
        plsc.subcore_barrier()

        pltpu.sync_copy(src_hbm.at[wid], src_v)
        pltpu.sync_copy(dst_hbm.at[wid], dst_v)
        pltpu.sync_copy(w_hbm.at[wid], w_v)

        def chunk(j, _):
            pltpu.async_copy(x_hbm.at[src_v.at[j]], rows_v, sem).wait()

            def scale(kk, _):
                wvec = w_v[j, pl.ds(kk * 16, 16)]
                for l in range(16):
                    wk = wvec[l]
                    row = kk * 16 + l
                    for cc in range(8):
                        sl = pl.ds(cc * 16, 16)
                        rows_v[row, sl] = rows_v[row, sl] * wk
                return 0
            lax.fori_loop(0, CH // 16, scale, 0)

            pltpu.sync_copy(rows_v, acc_sh.at[dst_v.at[j]], add=True)
            return 0
        lax.fori_loop(0, n_chunks, chunk, 0)

        plsc.subcore_barrier()

        pltpu.sync_copy(acc_sh.at[pl.ds(s * rows_per_sub, rows_per_sub)],
                        out_hbm.at[c, pl.ds(s * rows_per_sub, rows_per_sub)])

    return body(x, src3, dst3, w3)


def _tc_dense(p, W, b, n_nodes, d_feat, n_units, blk):
    def body(p_ref, w_ref, b_ref, o_ref):
        ah = p_ref[0] + p_ref[1]
        acc = jnp.dot(ah, w_ref[...], preferred_element_type=jnp.float32)
        o_ref[...] = jnp.maximum(acc + b_ref[...], 0.0)

    grid = (n_nodes // blk,)
    return pl.pallas_call(
        body,
        grid=grid,
        in_specs=[
            pl.BlockSpec((2, blk, d_feat), lambda i: (0, i, 0)),
            pl.BlockSpec((d_feat, n_units), lambda i: (0, 0)),
            pl.BlockSpec((1, n_units), lambda i: (0, 0)),
        ],
        out_specs=pl.BlockSpec((blk, n_units), lambda i: (i, 0)),
        out_shape=jax.ShapeDtypeStruct((n_nodes, n_units), jnp.float32),
    )(p, W, b.reshape(1, n_units))


def kernel(x, edge_index, edge_weight, W, b):
    n_nodes, d_feat = x.shape
    n_units = W.shape[1]
    n_edges = edge_weight.shape[0]

    src = edge_index[0].astype(jnp.int32)
    dst = edge_index[1].astype(jnp.int32)
    w = edge_weight.astype(jnp.float32)

    per_w = -(-n_edges // (NW * CH)) * CH
    n_chunks = per_w // CH
    e_pad = NW * per_w
    pad = e_pad - n_edges
    if pad:
        src = jnp.concatenate([src, jnp.zeros((pad,), jnp.int32)])
        dst = jnp.concatenate([dst, jnp.zeros((pad,), jnp.int32)])
        w = jnp.concatenate([w, jnp.zeros((pad,), jnp.float32)])
    src3 = src.reshape(NW, n_chunks, CH)
    dst3 = dst.reshape(NW, n_chunks, CH)
    w3 = w.reshape(NW, n_chunks, CH)

    n_acc = -(-n_nodes // (NS * CH)) * NS * CH
    p = _sc_aggregate(x, src3, dst3, w3, n_acc, d_feat, n_chunks)
    return _tc_dense(p, W, b, n_nodes, d_feat, n_units, blk=1000)
